# baseline (speedup 1.0000x reference)
import jax
import jax.numpy as jnp
from jax.experimental import pallas as pl

_C = 126
_FEAT = 40
_TOPK = 20
_G = 4
_OFFS = (0, 128, 384, 640)
_WIDTHS = (128, 192, 256, 320)
_TOTF = 960
_ISENT = -(2 ** 31)


def _conv_as_matmul(W, b):
    O, _, R, kw = W.shape
    Wd = 10 - kw + 1
    A = jnp.zeros((R, 10, O, Wd), jnp.float32)
    Wt = jnp.transpose(W[:, 0, :, :], (1, 2, 0))
    for w in range(Wd):
        A = A.at[:, w:w + kw, :, w].set(Wt)
    return A.reshape(R * 10, O * Wd), jnp.repeat(b, Wd)


def _pack_keys(raw):
    riota = jax.lax.broadcasted_iota(jnp.int32, (_C, _C), 0)
    bits = jax.lax.bitcast_convert_type(raw, jnp.int32)
    key0 = jnp.where(bits >= 0, bits, bits ^ jnp.int32(0x7FFFFFFF))
    key = (((key0 >> 8) << 7) | (jnp.int32(127) - riota)) \
        + jnp.int32(0x30000000)
    return jax.lax.bitcast_convert_type(key, jnp.float32)


def _batcher16():
    pairs = []
    p = 1
    while p < 16:
        k = p
        while k >= 1:
            for j in range(k % p, 16 - k, 2 * k):
                for i in range(0, min(k, 16 - j - k)):
                    if (i + j) // (p * 2) == (i + j + k) // (p * 2):
                        pairs.append((i + j, i + j + k))
            k //= 2
        p *= 2
    return pairs


_NET16 = _batcher16()


def _topk_threshold(keyf):
    kp = jax.lax.pad(keyf, jnp.float32(-1.0), [(0, 2, 0), (0, 0, 0)])
    s = [kp[8 * i:8 * (i + 1), :] for i in range(16)]
    for (i, j) in _NET16:
        hi = jnp.maximum(s[i], s[j])
        s[j] = jnp.minimum(s[i], s[j])
        s[i] = hi
    for t in range(_TOPK - 1):
        m = jnp.max(s[0], axis=0, keepdims=True)
        sel = s[0] == m
        depth = _TOPK - t - 1
        for i in range(min(15, depth)):
            s[i] = jnp.where(sel, s[i + 1], s[i])
        if depth > 15:
            s[15] = jnp.where(sel, -1.0, s[15])
    return jnp.max(s[0], axis=0, keepdims=True)


def _extract(keys):
    return [k >= _topk_threshold(k) for k in keys]


def _graph_body(xg, a_ref, bf_ref, branch_w, wend_ref, eye, eyef):
    feat = jnp.maximum(
        jnp.dot(xg, a_ref[...], preferred_element_type=jnp.float32)
        + bf_ref[0], 0.0)
    raws, hs = [], []
    for i in range(4):
        wbn, bbn, wg, bg = branch_w[i]
        f = feat[:, _OFFS[i]:_OFFS[i] + _WIDTHS[i]]
        xa = jnp.tanh(
            jnp.dot(f, wbn[...], preferred_element_type=jnp.float32)
            + bbn[0])
        hs.append(jnp.dot(f.astype(jnp.bfloat16),
                          wg[...].astype(jnp.bfloat16),
                          preferred_element_type=jnp.float32))
        raws.append(jax.lax.dot_general(
            xa, xa, (((1,), (1,)), ((), ())),
            preferred_element_type=jnp.float32))
    masks = _extract([_pack_keys(r) for r in raws])
    acc = [jnp.zeros((1, 32), jnp.float32) for _ in range(3)]
    for i in range(4):
        raw, h, mb = raws[i], hs[i], masks[i]
        bg = branch_w[i][3]
        e = jnp.exp(raw)
        st = e / jnp.sum(e, axis=0, keepdims=True)
        P = jnp.where(eye, 1.0, jnp.where(mb, st, 0.0))
        deg = jnp.sum(P, axis=0, keepdims=True)
        d = jax.lax.rsqrt(jnp.maximum(deg, 1.0))
        dcol = jax.lax.dot_general(eyef, d, (((1,), (1,)), ((), ())),
                                   preferred_element_type=jnp.float32)
        y = jax.lax.dot_general((P * d).astype(jnp.bfloat16),
                                (h * dcol).astype(jnp.bfloat16),
                                (((0,), (0,)), ((), ())),
                                preferred_element_type=jnp.float32)
        out = jnp.maximum(y + bg[0], 0.0)
        for t in range(3):
            p = out * wend_ref[t, i * _C:(i + 1) * _C, :]
            acc[t] = acc[t] + jnp.sum(p, axis=0, keepdims=True)
    lg = [jnp.sum(acc[t], axis=1, keepdims=True) for t in range(3)]
    return jnp.concatenate(lg, axis=1)


def _fused_kernel(x_ref, a_ref, bf_ref,
                  wbn0, bbn0, wg0, bg0, wbn1, bbn1, wg1, bg1,
                  wbn2, bbn2, wg2, bg2, wbn3, bbn3, wg3, bg3,
                  wend_ref, bend_ref, lo_ref, pr_ref):
    eye = (jax.lax.broadcasted_iota(jnp.int32, (_C, _C), 0)
           == jax.lax.broadcasted_iota(jnp.int32, (_C, _C), 1))
    eyef = jnp.where(eye, 1.0, 0.0)
    branch_w = ((wbn0, bbn0, wg0, bg0), (wbn1, bbn1, wg1, bg1),
                (wbn2, bbn2, wg2, bg2), (wbn3, bbn3, wg3, bg3))
    for gidx in range(_G):
        logits = _graph_body(x_ref[gidx], a_ref, bf_ref, branch_w,
                             wend_ref, eye, eyef) + bend_ref[...]
        lo_ref[gidx] = logits
        m = jnp.max(logits, axis=1, keepdims=True)
        ee = jnp.exp(logits - m)
        pr_ref[gidx] = ee / jnp.sum(ee, axis=1, keepdims=True)


def kernel(x, edge_index, batch, W0, b0, W1, b1, W2, b2, W3, b3,
           Wbn0, bbn0, Wg0, bg0, Wbn1, bbn1, Wg1, bg1,
           Wbn2, bbn2, Wg2, bg2, Wbn3, bbn3, Wg3, bg3,
           Wend, bend):
    n = x.shape[0] // _C
    xg = x.reshape(n, _C, _FEAT)
    A0, bf0 = _conv_as_matmul(W0, b0)
    A1, bf1 = _conv_as_matmul(W1, b1)
    A2, bf2 = _conv_as_matmul(W2, b2)
    A3, bf3 = _conv_as_matmul(W3, b3)
    zpadA = jnp.zeros((_FEAT, 64), jnp.float32)
    zpadb = jnp.zeros((64,), jnp.float32)
    A = jnp.concatenate([A0, A1, zpadA, A2, A3], axis=1)
    bf = jnp.concatenate([bf0, bf1, zpadb, bf2, bf3]).reshape(1, _TOTF)
    wend_r = Wend.reshape(4 * _C, 32, 3).transpose(2, 0, 1)

    def _full(shape):
        nd = len(shape)
        return pl.BlockSpec(shape, lambda g, _nd=nd: (0,) * _nd)

    weights = [A, bf,
               Wbn0, bbn0.reshape(1, 64), Wg0, bg0.reshape(1, 32),
               Wbn1, bbn1.reshape(1, 64), Wg1, bg1.reshape(1, 32),
               Wbn2, bbn2.reshape(1, 64), Wg2, bg2.reshape(1, 32),
               Wbn3, bbn3.reshape(1, 64), Wg3, bg3.reshape(1, 32),
               wend_r, bend.reshape(1, 3)]
    in_specs = [pl.BlockSpec((_G, _C, _FEAT), lambda g: (g, 0, 0))]
    in_specs += [_full(w.shape) for w in weights]
    out_specs = [pl.BlockSpec((_G, 1, 3), lambda g: (g, 0, 0))] * 2
    out_shape = [jax.ShapeDtypeStruct((n, 1, 3), jnp.float32)] * 2
    lo, pr = pl.pallas_call(
        _fused_kernel,
        grid=(n // _G,),
        in_specs=in_specs,
        out_specs=out_specs,
        out_shape=out_shape,
    )(xg, *weights)
    return lo.reshape(n, 3), pr.reshape(n, 3)


# speedup vs baseline: 1.0039x; 1.0039x over previous
import jax
import jax.numpy as jnp
from jax.experimental import pallas as pl

_C = 126
_FEAT = 40
_TOPK = 20
_G = 4
_OFFS = (0, 128, 384, 640)
_WIDTHS = (128, 192, 256, 320)
_TOTF = 960
_ISENT = -(2 ** 31)


def _conv_as_matmul(W, b):
    O, _, R, kw = W.shape
    Wd = 10 - kw + 1
    A = jnp.zeros((R, 10, O, Wd), jnp.float32)
    Wt = jnp.transpose(W[:, 0, :, :], (1, 2, 0))
    for w in range(Wd):
        A = A.at[:, w:w + kw, :, w].set(Wt)
    return A.reshape(R * 10, O * Wd), jnp.repeat(b, Wd)


def _pack_keys(raw):
    riota = jax.lax.broadcasted_iota(jnp.int32, (_C, _C), 0)
    bits = jax.lax.bitcast_convert_type(raw, jnp.int32)
    key0 = jnp.where(bits >= 0, bits, bits ^ jnp.int32(0x7FFFFFFF))
    key = (((key0 >> 8) << 7) | (jnp.int32(127) - riota)) \
        + jnp.int32(0x30000000)
    return jax.lax.bitcast_convert_type(key, jnp.float32)


def _batcher16():
    pairs = []
    p = 1
    while p < 16:
        k = p
        while k >= 1:
            for j in range(k % p, 16 - k, 2 * k):
                for i in range(0, min(k, 16 - j - k)):
                    if (i + j) // (p * 2) == (i + j + k) // (p * 2):
                        pairs.append((i + j, i + j + k))
            k //= 2
        p *= 2
    return pairs


_NET16 = _batcher16()


def _topk_threshold(keyf):
    kp = jax.lax.pad(keyf, jnp.float32(-1.0), [(0, 2, 0), (0, 0, 0)])
    s = [kp[8 * i:8 * (i + 1), :] for i in range(16)]
    for (i, j) in _NET16:
        hi = jnp.maximum(s[i], s[j])
        s[j] = jnp.minimum(s[i], s[j])
        s[i] = hi
    for t in range(_TOPK - 1):
        m = jnp.max(s[0], axis=0, keepdims=True)
        sel = s[0] == m
        depth = _TOPK - t - 1
        for i in range(min(15, depth)):
            s[i] = jnp.where(sel, s[i + 1], s[i])
        if depth > 15:
            s[15] = jnp.where(sel, -1.0, s[15])
    return jnp.max(s[0], axis=0, keepdims=True)


def _extract(keys):
    return [k >= _topk_threshold(k) for k in keys]


def _graph_body(xg, a_ref, bf_ref, branch_w, wend_ref, eye, eyef):
    feat = jnp.maximum(
        jnp.dot(xg, a_ref[...], preferred_element_type=jnp.float32)
        + bf_ref[0], 0.0)
    raws, hs = [], []
    for i in range(4):
        wbn, bbn, wg, bg = branch_w[i]
        f = feat[:, _OFFS[i]:_OFFS[i] + _WIDTHS[i]]
        xa = jnp.tanh(
            jnp.dot(f, wbn[...], preferred_element_type=jnp.float32)
            + bbn[0])
        hs.append(jnp.dot(f, wg[...], preferred_element_type=jnp.float32))
        raws.append(jax.lax.dot_general(
            xa, xa, (((1,), (1,)), ((), ())),
            preferred_element_type=jnp.float32))
    masks = _extract([_pack_keys(r) for r in raws])
    acc = [jnp.zeros((1, 32), jnp.float32) for _ in range(3)]
    for i in range(4):
        raw, h, mb = raws[i], hs[i], masks[i]
        bg = branch_w[i][3]
        e = jnp.exp(raw)
        st = e / jnp.sum(e, axis=0, keepdims=True)
        P = jnp.where(eye, 1.0, jnp.where(mb, st, 0.0))
        deg = jnp.sum(P, axis=0, keepdims=True)
        d = jax.lax.rsqrt(jnp.maximum(deg, 1.0))
        dcol = jax.lax.dot_general(eyef, d, (((1,), (1,)), ((), ())),
                                   preferred_element_type=jnp.float32)
        y = jax.lax.dot_general((P * d).astype(jnp.bfloat16),
                                (h * dcol).astype(jnp.bfloat16),
                                (((0,), (0,)), ((), ())),
                                preferred_element_type=jnp.float32)
        out = jnp.maximum(y + bg[0], 0.0)
        for t in range(3):
            p = out * wend_ref[t, i * _C:(i + 1) * _C, :]
            acc[t] = acc[t] + jnp.sum(p, axis=0, keepdims=True)
    lg = [jnp.sum(acc[t], axis=1, keepdims=True) for t in range(3)]
    return jnp.concatenate(lg, axis=1)


def _fused_kernel(x_ref, a_ref, bf_ref,
                  wbn0, bbn0, wg0, bg0, wbn1, bbn1, wg1, bg1,
                  wbn2, bbn2, wg2, bg2, wbn3, bbn3, wg3, bg3,
                  wend_ref, bend_ref, lo_ref, pr_ref):
    eye = (jax.lax.broadcasted_iota(jnp.int32, (_C, _C), 0)
           == jax.lax.broadcasted_iota(jnp.int32, (_C, _C), 1))
    eyef = jnp.where(eye, 1.0, 0.0)
    branch_w = ((wbn0, bbn0, wg0, bg0), (wbn1, bbn1, wg1, bg1),
                (wbn2, bbn2, wg2, bg2), (wbn3, bbn3, wg3, bg3))
    for gidx in range(_G):
        logits = _graph_body(x_ref[gidx], a_ref, bf_ref, branch_w,
                             wend_ref, eye, eyef) + bend_ref[...]
        lo_ref[gidx] = logits
        m = jnp.max(logits, axis=1, keepdims=True)
        ee = jnp.exp(logits - m)
        pr_ref[gidx] = ee / jnp.sum(ee, axis=1, keepdims=True)


def kernel(x, edge_index, batch, W0, b0, W1, b1, W2, b2, W3, b3,
           Wbn0, bbn0, Wg0, bg0, Wbn1, bbn1, Wg1, bg1,
           Wbn2, bbn2, Wg2, bg2, Wbn3, bbn3, Wg3, bg3,
           Wend, bend):
    n = x.shape[0] // _C
    xg = x.reshape(n, _C, _FEAT)
    A0, bf0 = _conv_as_matmul(W0, b0)
    A1, bf1 = _conv_as_matmul(W1, b1)
    A2, bf2 = _conv_as_matmul(W2, b2)
    A3, bf3 = _conv_as_matmul(W3, b3)
    zpadA = jnp.zeros((_FEAT, 64), jnp.float32)
    zpadb = jnp.zeros((64,), jnp.float32)
    A = jnp.concatenate([A0, A1, zpadA, A2, A3], axis=1)
    bf = jnp.concatenate([bf0, bf1, zpadb, bf2, bf3]).reshape(1, _TOTF)
    wend_r = Wend.reshape(4 * _C, 32, 3).transpose(2, 0, 1)

    def _full(shape):
        nd = len(shape)
        return pl.BlockSpec(shape, lambda g, _nd=nd: (0,) * _nd)

    weights = [A, bf,
               Wbn0, bbn0.reshape(1, 64), Wg0, bg0.reshape(1, 32),
               Wbn1, bbn1.reshape(1, 64), Wg1, bg1.reshape(1, 32),
               Wbn2, bbn2.reshape(1, 64), Wg2, bg2.reshape(1, 32),
               Wbn3, bbn3.reshape(1, 64), Wg3, bg3.reshape(1, 32),
               wend_r, bend.reshape(1, 3)]
    in_specs = [pl.BlockSpec((_G, _C, _FEAT), lambda g: (g, 0, 0))]
    in_specs += [_full(w.shape) for w in weights]
    out_specs = [pl.BlockSpec((_G, 1, 3), lambda g: (g, 0, 0))] * 2
    out_shape = [jax.ShapeDtypeStruct((n, 1, 3), jnp.float32)] * 2
    lo, pr = pl.pallas_call(
        _fused_kernel,
        grid=(n // _G,),
        in_specs=in_specs,
        out_specs=out_specs,
        out_shape=out_shape,
    )(xg, *weights)
    return lo.reshape(n, 3), pr.reshape(n, 3)


# speedup vs baseline: 1.0101x; 1.0062x over previous
import jax
import jax.numpy as jnp
from jax.experimental import pallas as pl

_C = 126
_FEAT = 40
_TOPK = 20
_G = 4
_OFFS = (0, 128, 384, 640)
_WIDTHS = (128, 192, 256, 320)
_TOTF = 960


def _conv_as_matmul(W, b):
    O, _, R, kw = W.shape
    Wd = 10 - kw + 1
    A = jnp.zeros((R, 10, O, Wd), jnp.float32)
    Wt = jnp.transpose(W[:, 0, :, :], (1, 2, 0))
    for w in range(Wd):
        A = A.at[:, w:w + kw, :, w].set(Wt)
    return A.reshape(R * 10, O * Wd), jnp.repeat(b, Wd)


def _pack_keys(raw):
    riota = jax.lax.broadcasted_iota(jnp.int32, (_C, _C), 0)
    bits = jax.lax.bitcast_convert_type(raw, jnp.int32)
    key0 = jnp.where(bits >= 0, bits, bits ^ jnp.int32(0x7FFFFFFF))
    key = (((key0 >> 8) << 7) | (jnp.int32(127) - riota)) \
        + jnp.int32(0x30000000)
    return jax.lax.bitcast_convert_type(key, jnp.float32)


def _batcher16():
    pairs = []
    p = 1
    while p < 16:
        k = p
        while k >= 1:
            for j in range(k % p, 16 - k, 2 * k):
                for i in range(0, min(k, 16 - j - k)):
                    if (i + j) // (p * 2) == (i + j + k) // (p * 2):
                        pairs.append((i + j, i + j + k))
            k //= 2
        p *= 2
    return pairs


_NET16 = _batcher16()


def _topk_threshold(keyf):
    kp = jax.lax.pad(keyf, jnp.float32(-1.0), [(0, 2, 0), (0, 0, 0)])
    s = [kp[8 * i:8 * (i + 1), :] for i in range(16)]
    for (i, j) in _NET16:
        hi = jnp.maximum(s[i], s[j])
        s[j] = jnp.minimum(s[i], s[j])
        s[i] = hi
    for t in range(_TOPK - 1):
        m = jnp.max(s[0], axis=0, keepdims=True)
        sel = s[0] == m
        depth = _TOPK - t - 1
        for i in range(min(15, depth)):
            s[i] = jnp.where(sel, s[i + 1], s[i])
        if depth > 15:
            s[15] = jnp.where(sel, -1.0, s[15])
    return jnp.max(s[0], axis=0, keepdims=True)


def _extract(keys):
    return [k >= _topk_threshold(k) for k in keys]


def _graph_body(xg, a_ref, bf_ref, branch_w, wend_ref, eye, eyef):
    feat = jnp.maximum(
        jnp.dot(xg, a_ref[...], preferred_element_type=jnp.float32)
        + bf_ref[0], 0.0)
    raws, hs = [], []
    for i in range(4):
        wbn, bbn, wg, bg = branch_w[i]
        f = feat[:, _OFFS[i]:_OFFS[i] + _WIDTHS[i]]
        xa = jnp.tanh(
            jnp.dot(f, wbn[...], preferred_element_type=jnp.float32)
            + bbn[0])
        hs.append(jnp.dot(f, wg[...], preferred_element_type=jnp.float32))
        raws.append(jax.lax.dot_general(
            xa, xa, (((1,), (1,)), ((), ())),
            preferred_element_type=jnp.float32))
    masks = _extract([_pack_keys(r) for r in raws])
    acc = [jnp.zeros((1, 32), jnp.float32) for _ in range(3)]
    for i in range(4):
        raw, h, mb = raws[i], hs[i], masks[i]
        bg = branch_w[i][3]
        e = jnp.exp(raw)
        st = e / jnp.sum(e, axis=0, keepdims=True)
        P = jnp.where(eye, 1.0, jnp.where(mb, st, 0.0))
        deg = jnp.sum(P, axis=0, keepdims=True)
        d = jax.lax.rsqrt(jnp.maximum(deg, 1.0))
        dcol = jax.lax.dot_general(eyef, d, (((1,), (1,)), ((), ())),
                                   preferred_element_type=jnp.float32)
        y = jax.lax.dot_general((P * d).astype(jnp.bfloat16),
                                (h * dcol).astype(jnp.bfloat16),
                                (((0,), (0,)), ((), ())),
                                preferred_element_type=jnp.float32)
        out = jnp.maximum(y + bg[0], 0.0)
        for t in range(3):
            p = out * wend_ref[t, i * _C:(i + 1) * _C, :]
            acc[t] = acc[t] + jnp.sum(p, axis=0, keepdims=True)
    lg = [jnp.sum(acc[t], axis=1, keepdims=True) for t in range(3)]
    return jnp.concatenate(lg, axis=1)


def _fused_kernel(x_ref, a_ref, bf_ref,
                  wbn0, bbn0, wg0, bg0, wbn1, bbn1, wg1, bg1,
                  wbn2, bbn2, wg2, bg2, wbn3, bbn3, wg3, bg3,
                  wend_ref, bend_ref, lo_ref, pr_ref):
    eye = (jax.lax.broadcasted_iota(jnp.int32, (_C, _C), 0)
           == jax.lax.broadcasted_iota(jnp.int32, (_C, _C), 1))
    eyef = jnp.where(eye, 1.0, 0.0)
    branch_w = ((wbn0, bbn0, wg0, bg0), (wbn1, bbn1, wg1, bg1),
                (wbn2, bbn2, wg2, bg2), (wbn3, bbn3, wg3, bg3))
    for gidx in range(_G):
        logits = _graph_body(x_ref[gidx], a_ref, bf_ref, branch_w,
                             wend_ref, eye, eyef) + bend_ref[...]
        lo_ref[gidx] = logits
        m = jnp.max(logits, axis=1, keepdims=True)
        ee = jnp.exp(logits - m)
        pr_ref[gidx] = ee / jnp.sum(ee, axis=1, keepdims=True)


def kernel(x, edge_index, batch, W0, b0, W1, b1, W2, b2, W3, b3,
           Wbn0, bbn0, Wg0, bg0, Wbn1, bbn1, Wg1, bg1,
           Wbn2, bbn2, Wg2, bg2, Wbn3, bbn3, Wg3, bg3,
           Wend, bend):
    n = x.shape[0] // _C
    xg = x.reshape(n, _C, _FEAT)
    A0, bf0 = _conv_as_matmul(W0, b0)
    A1, bf1 = _conv_as_matmul(W1, b1)
    A2, bf2 = _conv_as_matmul(W2, b2)
    A3, bf3 = _conv_as_matmul(W3, b3)
    zpadA = jnp.zeros((_FEAT, 64), jnp.float32)
    zpadb = jnp.zeros((64,), jnp.float32)
    A = jnp.concatenate([A0, A1, zpadA, A2, A3], axis=1)
    bf = jnp.concatenate([bf0, bf1, zpadb, bf2, bf3]).reshape(1, _TOTF)
    wend_r = Wend.reshape(4 * _C, 32, 3).transpose(2, 0, 1)

    def _full(shape):
        nd = len(shape)
        return pl.BlockSpec(shape, lambda g, _nd=nd: (0,) * _nd)

    weights = [A, bf,
               Wbn0, bbn0.reshape(1, 64), Wg0, bg0.reshape(1, 32),
               Wbn1, bbn1.reshape(1, 64), Wg1, bg1.reshape(1, 32),
               Wbn2, bbn2.reshape(1, 64), Wg2, bg2.reshape(1, 32),
               Wbn3, bbn3.reshape(1, 64), Wg3, bg3.reshape(1, 32),
               wend_r, bend.reshape(1, 3)]
    in_specs = [pl.BlockSpec((_G, _C, _FEAT), lambda g: (g, 0, 0))]
    in_specs += [_full(w.shape) for w in weights]
    out_specs = [pl.BlockSpec((_G, 1, 3), lambda g: (g, 0, 0))] * 2
    out_shape = [jax.ShapeDtypeStruct((n, 1, 3), jnp.float32)] * 2
    lo, pr = pl.pallas_call(
        _fused_kernel,
        grid=(n // _G,),
        in_specs=in_specs,
        out_specs=out_specs,
        out_shape=out_shape,
    )(xg, *weights)
    return lo.reshape(n, 3), pr.reshape(n, 3)
